# baseline (speedup 1.0000x reference)
import jax
import jax.numpy as jnp
from jax.experimental import pallas as pl
from jax.experimental.pallas import tpu as pltpu

EPS_BN = 1e-5
CDT = jnp.bfloat16


def _body(TN, P, C, H, x_ref, m_ref, ew_ref, wpre_ref, bpre_ref, w1a_ref,
          w1b_ref, b1_ref, w2_ref, b2_ref, wo1_ref, bo1_ref, wo2_ref,
          bo2_ref, out_ref):
    PH = P * H
    PC = P * C
    KT = 256
    NCH = PH // KT

    x = x_ref[...]
    mf = m_ref[...].astype(CDT)

    mw = jnp.dot(mf, ew_ref[...], preferred_element_type=jnp.float32) \
        .astype(CDT)

    def bd_mm(v, k4):
        return jnp.concatenate(
            [jnp.dot(v[:, c * KT:(c + 1) * KT], k4,
                     preferred_element_type=jnp.float32)
             for c in range(NCH)], axis=1)

    def pool(v):
        w = PH
        while w > H:
            w //= 2
            v = jnp.maximum(v[:, :w], v[:, w:2 * w])
        return v

    def lane_tile(v):
        w = H
        while w < PH:
            v = jnp.concatenate([v, v], axis=1)
            w *= 2
        return v

    half = PC // 2
    h = jnp.concatenate(
        [jnp.dot(x[:, :half], wpre_ref[...],
                 preferred_element_type=jnp.float32),
         jnp.dot(x[:, half:], wpre_ref[...],
                 preferred_element_type=jnp.float32)], axis=1)
    feat = (jnp.maximum(h + bpre_ref[...], 0.0).astype(CDT)) * mw
    pooled = pool(feat)
    a = bd_mm(feat, w1a_ref[...])
    bw = jnp.dot(pooled, w1b_ref[...], preferred_element_type=jnp.float32) \
        + b1_ref[...]
    y3 = jnp.maximum(a + lane_tile(bw), 0.0).astype(CDT)
    y2 = (jnp.maximum(bd_mm(y3, w2_ref[...]) + b2_ref[...], 0.0)
          .astype(CDT)) * mw
    out = pool(y2).astype(jnp.float32)

    v = m_ref[...].astype(jnp.float32)
    w = P
    while w > 1:
        w //= 2
        v = jnp.maximum(v[:, :w], v[:, w:2 * w])
    valid = v

    z = jnp.maximum(
        jnp.dot(out, wo1_ref[...], preferred_element_type=jnp.float32)
        + bo1_ref[...], 0.0)
    z2 = jnp.dot(z, wo2_ref[...], preferred_element_type=jnp.float32) \
        + bo2_ref[...]
    out_ref[...] = jnp.where(valid > 0.0, z2, 0.0)


def kernel(polylines, polylines_mask, W_pre, g_pre, b_pre, W1, g1, b1,
           W2, g2, b2, W_out1, b_out1, W_out2, b_out2):
    B, N, P, C = polylines.shape
    H = W_pre.shape[1]
    O = W_out2.shape[1]
    BN = B * N
    TN = 1024
    PH = P * H

    inv = 1.0 / jnp.sqrt(1.0 + EPS_BN)
    wpre_s = W_pre * (g_pre * inv)[None, :]
    wtop = jnp.kron(jnp.eye(P // 2, dtype=jnp.float32), wpre_s)
    w1s = W1 * (g1 * inv)[None, :]
    w1a = jnp.kron(jnp.eye(4, dtype=CDT), w1s[:H].astype(CDT))
    w1b = w1s[H:].astype(CDT)
    w2s = jnp.kron(jnp.eye(4, dtype=CDT),
                   (W2 * (g2 * inv)[None, :]).astype(CDT))
    bpre = jnp.tile(b_pre, P).reshape(1, PH)
    b2t = jnp.tile(b2, P).reshape(1, PH)
    ew = jnp.kron(jnp.eye(P, dtype=CDT),
                  jnp.ones((1, H), dtype=CDT))

    row = lambda i: (i, 0)
    full = lambda i: (0, 0)

    NCHUNK = 2
    BNc = BN // NCHUNK
    Bc = B // NCHUNK

    def run_chunk(pslice, mslice):
        xp = pslice.reshape(BNc, P * C)
        mm = mslice.reshape(BNc, P)
        return pl.pallas_call(
            lambda *refs: _body(TN, P, C, H, *refs),
            grid=(BNc // TN,),
            in_specs=[
                pl.BlockSpec((TN, P * C), row),
                pl.BlockSpec((TN, P), row),
                pl.BlockSpec((P, PH), full),
                pl.BlockSpec(wtop.shape, full),
                pl.BlockSpec((1, PH), full),
                pl.BlockSpec((256, 256), full),
                pl.BlockSpec((H, H), full),
                pl.BlockSpec((1, H), full),
                pl.BlockSpec((256, 256), full),
                pl.BlockSpec((1, PH), full),
                pl.BlockSpec((H, H), full),
                pl.BlockSpec((1, H), full),
                pl.BlockSpec((H, O), full),
                pl.BlockSpec((1, O), full),
            ],
            out_specs=pl.BlockSpec((TN, O), row),
            out_shape=jax.ShapeDtypeStruct((BNc, O), jnp.float32),
            compiler_params=pltpu.CompilerParams(
                dimension_semantics=("parallel",)),
        )(xp, mm, ew, wtop, bpre, w1a, w1b, b1.reshape(1, H), w2s, b2t,
          W_out1, b_out1.reshape(1, H), W_out2, b_out2.reshape(1, O))

    outs = [run_chunk(polylines[c * Bc:(c + 1) * Bc],
                      polylines_mask[c * Bc:(c + 1) * Bc])
            for c in range(NCHUNK)]
    return jnp.concatenate(outs, axis=0).reshape(B, N, O)


# speedup vs baseline: 1.1567x; 1.1567x over previous
import jax
import jax.numpy as jnp
from jax.experimental import pallas as pl
from jax.experimental.pallas import tpu as pltpu

EPS_BN = 1e-5
CDT = jnp.bfloat16


def _body(TN, P, C, H, x_ref, m_ref, ew_ref, wpre_ref, bpre_ref, w1a_ref,
          w1b_ref, b1_ref, w2_ref, b2_ref, wo1_ref, bo1_ref, wo2_ref,
          bo2_ref, out_ref):
    PH = P * H
    PC = P * C
    KT = 256
    NCH = PH // KT

    x = x_ref[...]
    mf = m_ref[...].astype(CDT)

    mw = jnp.dot(mf, ew_ref[...], preferred_element_type=jnp.float32) \
        .astype(CDT)

    def bd_mm(v, k4):
        return jnp.concatenate(
            [jnp.dot(v[:, c * KT:(c + 1) * KT], k4,
                     preferred_element_type=jnp.float32)
             for c in range(NCH)], axis=1)

    def pool(v):
        w = PH
        while w > H:
            w //= 2
            v = jnp.maximum(v[:, :w], v[:, w:2 * w])
        return v

    def lane_tile(v):
        w = H
        while w < PH:
            v = jnp.concatenate([v, v], axis=1)
            w *= 2
        return v

    half = PC // 2
    h = jnp.concatenate(
        [jnp.dot(x[:, :half], wpre_ref[...],
                 preferred_element_type=jnp.float32),
         jnp.dot(x[:, half:], wpre_ref[...],
                 preferred_element_type=jnp.float32)], axis=1)
    feat = (jnp.maximum(h + bpre_ref[...], 0.0).astype(CDT)) * mw
    pooled = pool(feat)
    a = bd_mm(feat, w1a_ref[...])
    bw = jnp.dot(pooled, w1b_ref[...], preferred_element_type=jnp.float32) \
        + b1_ref[...]
    y3 = jnp.maximum(a + lane_tile(bw), 0.0).astype(CDT)
    y2 = (jnp.maximum(bd_mm(y3, w2_ref[...]) + b2_ref[...], 0.0)
          .astype(CDT)) * mw
    out = pool(y2).astype(jnp.float32)

    v = m_ref[...].astype(jnp.float32)
    w = P
    while w > 1:
        w //= 2
        v = jnp.maximum(v[:, :w], v[:, w:2 * w])
    valid = v

    z = jnp.maximum(
        jnp.dot(out, wo1_ref[...], preferred_element_type=jnp.float32)
        + bo1_ref[...], 0.0)
    z2 = jnp.dot(z, wo2_ref[...], preferred_element_type=jnp.float32) \
        + bo2_ref[...]
    out_ref[...] = jnp.where(valid > 0.0, z2, 0.0)


def kernel(polylines, polylines_mask, W_pre, g_pre, b_pre, W1, g1, b1,
           W2, g2, b2, W_out1, b_out1, W_out2, b_out2):
    B, N, P, C = polylines.shape
    H = W_pre.shape[1]
    O = W_out2.shape[1]
    BN = B * N
    TN = 2048
    PH = P * H

    inv = 1.0 / jnp.sqrt(1.0 + EPS_BN)
    wpre_s = W_pre * (g_pre * inv)[None, :]
    wtop = jnp.kron(jnp.eye(P // 2, dtype=jnp.float32), wpre_s)
    w1s = W1 * (g1 * inv)[None, :]
    w1a = jnp.kron(jnp.eye(4, dtype=CDT), w1s[:H].astype(CDT))
    w1b = w1s[H:].astype(CDT)
    w2s = jnp.kron(jnp.eye(4, dtype=CDT),
                   (W2 * (g2 * inv)[None, :]).astype(CDT))
    bpre = jnp.tile(b_pre, P).reshape(1, PH)
    b2t = jnp.tile(b2, P).reshape(1, PH)
    ew = jnp.kron(jnp.eye(P, dtype=CDT),
                  jnp.ones((1, H), dtype=CDT))

    row = lambda i: (i, 0)
    full = lambda i: (0, 0)

    def run_chunk(pslice, mslice):
        xp = pslice.reshape(BN, P * C)
        mm = mslice.reshape(BN, P)
        return pl.pallas_call(
            lambda *refs: _body(TN, P, C, H, *refs),
            grid=(BN // TN,),
            in_specs=[
                pl.BlockSpec((TN, P * C), row),
                pl.BlockSpec((TN, P), row),
                pl.BlockSpec((P, PH), full),
                pl.BlockSpec(wtop.shape, full),
                pl.BlockSpec((1, PH), full),
                pl.BlockSpec((256, 256), full),
                pl.BlockSpec((H, H), full),
                pl.BlockSpec((1, H), full),
                pl.BlockSpec((256, 256), full),
                pl.BlockSpec((1, PH), full),
                pl.BlockSpec((H, H), full),
                pl.BlockSpec((1, H), full),
                pl.BlockSpec((H, O), full),
                pl.BlockSpec((1, O), full),
            ],
            out_specs=pl.BlockSpec((TN, O), row),
            out_shape=jax.ShapeDtypeStruct((BN, O), jnp.float32),
            compiler_params=pltpu.CompilerParams(
                dimension_semantics=("parallel",)),
        )(xp, mm, ew, wtop, bpre, w1a, w1b, b1.reshape(1, H), w2s, b2t,
          W_out1, b_out1.reshape(1, H), W_out2, b_out2.reshape(1, O))

    return run_chunk(polylines, polylines_mask).reshape(B, N, O)


# speedup vs baseline: 1.2401x; 1.0721x over previous
import jax
import jax.numpy as jnp
from jax.experimental import pallas as pl
from jax.experimental.pallas import tpu as pltpu

EPS_BN = 1e-5
CDT = jnp.bfloat16


def _body(TN, P, C, H, x_ref, m_ref, ew_ref, wpre_ref, bpre_ref, w1a_ref,
          w1b_ref, b1_ref, w2_ref, b2_ref, wo1_ref, bo1_ref, wo2_ref,
          bo2_ref, out_ref):
    PH = P * H
    PC = P * C
    KT = 256
    NCH = PH // KT

    x = x_ref[...]
    mf = m_ref[...].astype(CDT)

    mw = jnp.dot(mf, ew_ref[...], preferred_element_type=jnp.float32) \
        .astype(CDT)

    def bd_mm(v, k4):
        return jnp.concatenate(
            [jnp.dot(v[:, c * KT:(c + 1) * KT], k4,
                     preferred_element_type=jnp.float32)
             for c in range(NCH)], axis=1)

    def pool(v):
        w = PH
        while w > H:
            w //= 2
            v = jnp.maximum(v[:, :w], v[:, w:2 * w])
        return v

    def lane_tile(v):
        w = H
        while w < PH:
            v = jnp.concatenate([v, v], axis=1)
            w *= 2
        return v

    half = PC // 2
    h = jnp.concatenate(
        [jnp.dot(x[:, :half], wpre_ref[...],
                 preferred_element_type=jnp.float32),
         jnp.dot(x[:, half:], wpre_ref[...],
                 preferred_element_type=jnp.float32)], axis=1)
    feat = (jnp.maximum(h + bpre_ref[...], 0.0).astype(CDT)) * mw
    pooled = pool(feat)
    a = bd_mm(feat, w1a_ref[...])
    bw = jnp.dot(pooled, w1b_ref[...], preferred_element_type=jnp.float32) \
        + b1_ref[...]
    y3 = jnp.maximum(a + lane_tile(bw), 0.0).astype(CDT)
    y2 = (jnp.maximum(bd_mm(y3, w2_ref[...]) + b2_ref[...], 0.0)
          .astype(CDT)) * mw
    out = pool(y2).astype(jnp.float32)

    v = m_ref[...].astype(jnp.float32)
    w = P
    while w > 1:
        w //= 2
        v = jnp.maximum(v[:, :w], v[:, w:2 * w])
    valid = v

    z = jnp.maximum(
        jnp.dot(out, wo1_ref[...], preferred_element_type=jnp.float32)
        + bo1_ref[...], 0.0)
    z2 = jnp.dot(z, wo2_ref[...], preferred_element_type=jnp.float32) \
        + bo2_ref[...]
    out_ref[...] = jnp.where(valid > 0.0, z2, 0.0)


def kernel(polylines, polylines_mask, W_pre, g_pre, b_pre, W1, g1, b1,
           W2, g2, b2, W_out1, b_out1, W_out2, b_out2):
    B, N, P, C = polylines.shape
    H = W_pre.shape[1]
    O = W_out2.shape[1]
    BN = B * N
    TN = 2048
    PH = P * H

    inv = 1.0 / jnp.sqrt(1.0 + EPS_BN)
    wpre_s = (W_pre * (g_pre * inv)[None, :]).astype(CDT)
    wtop = jnp.kron(jnp.eye(P // 2, dtype=CDT), wpre_s)
    w1s = W1 * (g1 * inv)[None, :]
    w1a = jnp.kron(jnp.eye(4, dtype=CDT), w1s[:H].astype(CDT))
    w1b = w1s[H:].astype(CDT)
    w2s = jnp.kron(jnp.eye(4, dtype=CDT),
                   (W2 * (g2 * inv)[None, :]).astype(CDT))
    bpre = jnp.tile(b_pre, P).reshape(1, PH)
    b2t = jnp.tile(b2, P).reshape(1, PH)
    ew = jnp.kron(jnp.eye(P, dtype=CDT),
                  jnp.ones((1, H), dtype=CDT))

    row = lambda i: (i, 0)
    full = lambda i: (0, 0)

    def run_chunk(pslice, mslice):
        xp = pslice.reshape(BN, P * C).astype(CDT)
        mm = mslice.reshape(BN, P)
        return pl.pallas_call(
            lambda *refs: _body(TN, P, C, H, *refs),
            grid=(BN // TN,),
            in_specs=[
                pl.BlockSpec((TN, P * C), row),
                pl.BlockSpec((TN, P), row),
                pl.BlockSpec((P, PH), full),
                pl.BlockSpec(wtop.shape, full),
                pl.BlockSpec((1, PH), full),
                pl.BlockSpec((256, 256), full),
                pl.BlockSpec((H, H), full),
                pl.BlockSpec((1, H), full),
                pl.BlockSpec((256, 256), full),
                pl.BlockSpec((1, PH), full),
                pl.BlockSpec((H, H), full),
                pl.BlockSpec((1, H), full),
                pl.BlockSpec((H, O), full),
                pl.BlockSpec((1, O), full),
            ],
            out_specs=pl.BlockSpec((TN, O), row),
            out_shape=jax.ShapeDtypeStruct((BN, O), jnp.float32),
            compiler_params=pltpu.CompilerParams(
                dimension_semantics=("parallel",)),
        )(xp, mm, ew, wtop, bpre, w1a, w1b, b1.reshape(1, H), w2s, b2t,
          W_out1, b_out1.reshape(1, H), W_out2, b_out2.reshape(1, O))

    return run_chunk(polylines, polylines_mask).reshape(B, N, O)
